# Initial kernel scaffold; baseline (speedup 1.0000x reference)
import jax
import jax.numpy as jnp
from jax.experimental import pallas as pl


def kernel(edge_index, node_latent, fij, tij, Wm1, bm1, Wm2, bm2, Wi1, bi1, Wi2, bi2, Wd1, bd1, Wd2, bd2):
    raise NotImplementedError("write your pallas kernel here")


# speedup vs baseline: 1.3368x; 1.3368x over previous
import functools

import jax
import jax.numpy as jnp
from jax import lax
from jax.experimental import pallas as pl
from jax.experimental.pallas import tpu as pltpu
from jax.experimental.pallas import tpu_sc as plsc

LATENT = 128
E_CHUNK = 1024
N_STREAM = 8
IDX_W = 128
ROW_W = 8
NC = 2
NS = 16
NW = NC * NS
STRIPE = 6272
N_PAD = STRIPE * NS


def _sc_scatter_partials(recv, fij, tij, zrows):
    E = fij.shape[0]
    G = E // E_CHUNK
    mesh = plsc.VectorSubcoreMesh(core_axis_name="c", subcore_axis_name="s")

    @functools.partial(
        pl.kernel,
        out_type=jax.ShapeDtypeStruct((NC, N_PAD, ROW_W), jnp.float32),
        mesh=mesh,
        scratch_types=[
            pltpu.VMEM_SHARED((N_PAD, ROW_W), jnp.float32),
            pltpu.VMEM((E_CHUNK,), jnp.int32),
            pltpu.VMEM((E_CHUNK, ROW_W), jnp.float32),
            pltpu.SemaphoreType.DMA,
        ],
        compiler_params=pltpu.CompilerParams(use_tc_tiling_on_sc=False),
    )
    def k(recv_hbm, fij_hbm, tij_hbm, z_hbm, p_hbm, acc, idx_v, comb_v, sem):
        c = lax.axis_index("c")
        s = lax.axis_index("s")
        tid = s * NC + c

        pltpu.sync_copy(z_hbm, acc.at[pl.ds(s * STRIPE, STRIPE), :])
        pltpu.sync_copy(z_hbm.at[pl.ds(0, E_CHUNK), :], comb_v)
        plsc.subcore_barrier()

        n_k = (G - tid + NW - 1) // NW

        def body(k_i, carry):
            g = tid + k_i * NW
            pltpu.sync_copy(recv_hbm.at[pl.ds(g * E_CHUNK, E_CHUNK)], idx_v)
            pltpu.sync_copy(fij_hbm.at[pl.ds(g * E_CHUNK, E_CHUNK), :],
                            comb_v.at[:, 0:3])
            pltpu.sync_copy(tij_hbm.at[pl.ds(g * E_CHUNK, E_CHUNK), :],
                            comb_v.at[:, 4:7])
            cps = []
            for j in range(N_STREAM):
                cps.append(pltpu.async_copy(
                    comb_v.at[pl.ds(j * IDX_W, IDX_W), :],
                    acc.at[idx_v.at[pl.ds(j * IDX_W, IDX_W)]],
                    sem, add=True))
            for cp in cps:
                cp.wait()
            return carry

        lax.fori_loop(0, n_k, body, 0)
        plsc.subcore_barrier()

        pltpu.sync_copy(acc.at[pl.ds(s * STRIPE, STRIPE), :],
                        p_hbm.at[c, pl.ds(s * STRIPE, STRIPE), :])

    return k(recv, fij, tij, zrows)


def _tc_body(x_r, w1_r, b1_r, wm2_r, bm2_r, wi2_r, bi2_r, wd2_r, bd2_r,
             p0_r, p1_r, dv_r, dw_r):
    x = x_r[...]
    h = jnp.maximum(
        jnp.dot(x, w1_r[...], preferred_element_type=jnp.float32) + b1_r[...],
        0.0)
    m = jnp.dot(h[:, :LATENT], wm2_r[...],
                preferred_element_type=jnp.float32) + bm2_r[...]
    i = jnp.dot(h[:, LATENT:2 * LATENT], wi2_r[...],
                preferred_element_type=jnp.float32) + bi2_r[...]
    d = jnp.dot(h[:, 2 * LATENT:], wd2_r[...],
                preferred_element_type=jnp.float32) + bd2_r[...]
    p = p0_r[...] + p1_r[...]
    f = p[:, 0:3]
    t = p[:, 4:7]
    dv_r[...] = m * f + d
    dw_r[...] = i * t


def _tc_decode(x, w1c, b1c, wm2, bm2, wi2, bi2, wd2, bd2, p0, p1):
    n = x.shape[0]
    blk = 4000
    grid = n // blk
    full = lambda shape: pl.BlockSpec(shape, lambda i: (0, 0))
    row = lambda w: pl.BlockSpec((blk, w), lambda i: (i, 0))
    return pl.pallas_call(
        _tc_body,
        grid=(grid,),
        in_specs=[
            row(LATENT),
            full((LATENT, 3 * LATENT)),
            full((1, 3 * LATENT)),
            full((LATENT, 1)),
            full((1, 1)),
            full((LATENT, 1)),
            full((1, 1)),
            full((LATENT, 3)),
            full((1, 3)),
            row(ROW_W), row(ROW_W),
        ],
        out_specs=[row(3), row(3)],
        out_shape=[
            jax.ShapeDtypeStruct((n, 3), jnp.float32),
            jax.ShapeDtypeStruct((n, 3), jnp.float32),
        ],
    )(x, w1c, b1c, wm2, bm2, wi2, bi2, wd2, bd2, p0, p1)


def kernel(edge_index, node_latent, fij, tij, Wm1, bm1, Wm2, bm2,
           Wi1, bi1, Wi2, bi2, Wd1, bd1, Wd2, bd2):
    n = node_latent.shape[0]
    recv = edge_index[1].astype(jnp.int32)
    zrows = jnp.zeros((STRIPE, ROW_W), jnp.float32)

    p = _sc_scatter_partials(recv, fij, tij, zrows)

    w1c = jnp.concatenate([Wm1, Wi1, Wd1], axis=1)
    b1c = jnp.concatenate([bm1, bi1, bd1]).reshape(1, 3 * LATENT)
    dv, dw = _tc_decode(
        node_latent, w1c, b1c,
        Wm2, bm2.reshape(1, 1), Wi2, bi2.reshape(1, 1),
        Wd2, bd2.reshape(1, 3),
        p[0, :n], p[1, :n])
    return (dv, dw)


# speedup vs baseline: 6.8092x; 5.0937x over previous
import functools

import jax
import jax.numpy as jnp
from jax import lax
from jax.experimental import pallas as pl
from jax.experimental.pallas import tpu as pltpu
from jax.experimental.pallas import tpu_sc as plsc

LATENT = 128
E_CHUNK = 1024
N_STREAM = 8
IDX_W = 128
ROW_W = 8
NC = 2
NS = 16
NW = NC * NS
STRIPE = 6272
N_PAD = STRIPE * NS


def _pack_body(f_r, t_r, o_r):
    f = f_r[...]
    t = t_r[...]
    z = jnp.zeros((f.shape[0], 1), jnp.float32)
    o_r[...] = jnp.concatenate([f, z, t, z], axis=1)


def _tc_pack(fij, tij):
    e = fij.shape[0]
    blk = 4000
    grid = e // blk
    return pl.pallas_call(
        _pack_body,
        grid=(grid,),
        in_specs=[
            pl.BlockSpec((blk, 3), lambda i: (i, 0)),
            pl.BlockSpec((blk, 3), lambda i: (i, 0)),
        ],
        out_specs=pl.BlockSpec((blk, ROW_W), lambda i: (i, 0)),
        out_shape=jax.ShapeDtypeStruct((e, ROW_W), jnp.float32),
    )(fij, tij)


def _sc_scatter_partials(recv, ft, zrows):
    E = ft.shape[0]
    G = E // E_CHUNK
    mesh = plsc.VectorSubcoreMesh(core_axis_name="c", subcore_axis_name="s")

    @functools.partial(
        pl.kernel,
        out_type=jax.ShapeDtypeStruct((NC, N_PAD, ROW_W), jnp.float32),
        mesh=mesh,
        scratch_types=[
            pltpu.VMEM_SHARED((N_PAD, ROW_W), jnp.float32),
            pltpu.VMEM((2, E_CHUNK), jnp.int32),
            pltpu.VMEM((2, E_CHUNK, ROW_W), jnp.float32),
            pltpu.SemaphoreType.DMA,
            pltpu.SemaphoreType.DMA,
        ],
        compiler_params=pltpu.CompilerParams(use_tc_tiling_on_sc=False),
    )
    def k(recv_hbm, ft_hbm, z_hbm, p_hbm, acc, idx_v, row_v, sem_in, sem_sc):
        c = lax.axis_index("c")
        s = lax.axis_index("s")
        tid = s * NC + c

        pltpu.sync_copy(z_hbm, acc.at[pl.ds(s * STRIPE, STRIPE), :])
        plsc.subcore_barrier()

        n_k = (G - tid + NW - 1) // NW

        def stage(k_i, b):
            g = tid + k_i * NW
            cp_i = pltpu.make_async_copy(
                recv_hbm.at[pl.ds(g * E_CHUNK, E_CHUNK)], idx_v.at[b],
                sem_in)
            cp_i.start()
            cp_r = pltpu.make_async_copy(
                ft_hbm.at[pl.ds(g * E_CHUNK, E_CHUNK), :], row_v.at[b],
                sem_in)
            cp_r.start()
            return cp_i, cp_r

        pr = stage(0, 0)

        def body(k_i, carry):
            b = lax.rem(k_i, 2)
            pltpu.make_async_copy(
                recv_hbm.at[pl.ds(0, E_CHUNK)], idx_v.at[b], sem_in).wait()
            pltpu.make_async_copy(
                ft_hbm.at[pl.ds(0, E_CHUNK), :], row_v.at[b], sem_in).wait()

            @pl.when(k_i + 1 < n_k)
            def _():
                stage(k_i + 1, 1 - b)

            cps = []
            for j in range(N_STREAM):
                cps.append(pltpu.async_copy(
                    row_v.at[b, pl.ds(j * IDX_W, IDX_W), :],
                    acc.at[idx_v.at[b, pl.ds(j * IDX_W, IDX_W)]],
                    sem_sc, add=True))
            for cp in cps:
                cp.wait()
            return carry

        lax.fori_loop(0, n_k, body, 0, unroll=False)
        plsc.subcore_barrier()

        pltpu.sync_copy(acc.at[pl.ds(s * STRIPE, STRIPE), :],
                        p_hbm.at[c, pl.ds(s * STRIPE, STRIPE), :])

    return k(recv, ft, zrows)


def _tc_body(x_r, w1_r, b1_r, wm2_r, bm2_r, wi2_r, bi2_r, wd2_r, bd2_r,
             p0_r, p1_r, dv_r, dw_r):
    x = x_r[...]
    h = jnp.maximum(
        jnp.dot(x, w1_r[...], preferred_element_type=jnp.float32) + b1_r[...],
        0.0)
    m = jnp.dot(h[:, :LATENT], wm2_r[...],
                preferred_element_type=jnp.float32) + bm2_r[...]
    i = jnp.dot(h[:, LATENT:2 * LATENT], wi2_r[...],
                preferred_element_type=jnp.float32) + bi2_r[...]
    d = jnp.dot(h[:, 2 * LATENT:], wd2_r[...],
                preferred_element_type=jnp.float32) + bd2_r[...]
    p = p0_r[...] + p1_r[...]
    f = p[:, 0:3]
    t = p[:, 4:7]
    dv_r[...] = m * f + d
    dw_r[...] = i * t


def _tc_decode(x, w1c, b1c, wm2, bm2, wi2, bi2, wd2, bd2, p0, p1):
    n = x.shape[0]
    blk = 4000
    grid = n // blk
    full = lambda shape: pl.BlockSpec(shape, lambda i: (0, 0))
    row = lambda w: pl.BlockSpec((blk, w), lambda i: (i, 0))
    return pl.pallas_call(
        _tc_body,
        grid=(grid,),
        in_specs=[
            row(LATENT),
            full((LATENT, 3 * LATENT)),
            full((1, 3 * LATENT)),
            full((LATENT, 1)),
            full((1, 1)),
            full((LATENT, 1)),
            full((1, 1)),
            full((LATENT, 3)),
            full((1, 3)),
            row(ROW_W), row(ROW_W),
        ],
        out_specs=[row(3), row(3)],
        out_shape=[
            jax.ShapeDtypeStruct((n, 3), jnp.float32),
            jax.ShapeDtypeStruct((n, 3), jnp.float32),
        ],
    )(x, w1c, b1c, wm2, bm2, wi2, bi2, wd2, bd2, p0, p1)


def kernel(edge_index, node_latent, fij, tij, Wm1, bm1, Wm2, bm2,
           Wi1, bi1, Wi2, bi2, Wd1, bd1, Wd2, bd2):
    n = node_latent.shape[0]
    recv = edge_index[1].astype(jnp.int32)
    zrows = jnp.zeros((STRIPE, ROW_W), jnp.float32)

    ft = _tc_pack(fij, tij)
    p = _sc_scatter_partials(recv, ft, zrows)

    w1c = jnp.concatenate([Wm1, Wi1, Wd1], axis=1)
    b1c = jnp.concatenate([bm1, bi1, bd1]).reshape(1, 3 * LATENT)
    dv, dw = _tc_decode(
        node_latent, w1c, b1c,
        Wm2, bm2.reshape(1, 1), Wi2, bi2.reshape(1, 1),
        Wd2, bd2.reshape(1, 3),
        p[0, :n], p[1, :n])
    return (dv, dw)


# speedup vs baseline: 17.8238x; 2.6176x over previous
import functools

import jax
import jax.numpy as jnp
from jax import lax
from jax.experimental import pallas as pl
from jax.experimental.pallas import tpu as pltpu
from jax.experimental.pallas import tpu_sc as plsc

LATENT = 128
E_CHUNK = 1024
N_STREAM = 8
IDX_W = 128
ROW_W = 8
NC = 2
NS = 16
NW = NC * NS
STRIPE = 6272
N_PAD = STRIPE * NS


def _sc_scatter_partials(recv, ft, zrows):
    E = ft.shape[0] * IDX_W
    G = E // E_CHUNK
    mesh = plsc.VectorSubcoreMesh(core_axis_name="c", subcore_axis_name="s")

    @functools.partial(
        pl.kernel,
        out_type=jax.ShapeDtypeStruct((NC, N_PAD, ROW_W), jnp.float32),
        mesh=mesh,
        scratch_types=[
            pltpu.VMEM_SHARED((N_PAD, ROW_W), jnp.float32),
            pltpu.VMEM((2, E_CHUNK), jnp.int32),
            pltpu.VMEM((2, N_STREAM, IDX_W, ROW_W), jnp.float32),
            pltpu.SemaphoreType.DMA,
            pltpu.SemaphoreType.DMA,
        ],
        compiler_params=pltpu.CompilerParams(use_tc_tiling_on_sc=False),
    )
    def k(recv_hbm, ft_hbm, z_hbm, p_hbm, acc, idx_v, row_v, sem_in, sem_sc):
        c = lax.axis_index("c")
        s = lax.axis_index("s")
        tid = s * NC + c

        pltpu.sync_copy(z_hbm, acc.at[pl.ds(s * STRIPE, STRIPE), :])
        plsc.subcore_barrier()

        n_k = (G - tid + NW - 1) // NW

        def stage(k_i, b):
            g = tid + k_i * NW
            cp_i = pltpu.make_async_copy(
                recv_hbm.at[pl.ds(g * E_CHUNK, E_CHUNK)], idx_v.at[b],
                sem_in)
            cp_i.start()
            cp_r = pltpu.make_async_copy(
                ft_hbm.at[pl.ds(g * N_STREAM, N_STREAM), :, :], row_v.at[b],
                sem_in)
            cp_r.start()
            return cp_i, cp_r

        pr = stage(0, 0)

        def body(k_i, carry):
            b = lax.rem(k_i, 2)
            pltpu.make_async_copy(
                recv_hbm.at[pl.ds(0, E_CHUNK)], idx_v.at[b], sem_in).wait()
            pltpu.make_async_copy(
                ft_hbm.at[pl.ds(0, N_STREAM), :, :], row_v.at[b], sem_in).wait()

            @pl.when(k_i + 1 < n_k)
            def _():
                stage(k_i + 1, 1 - b)

            cps = []
            for j in range(N_STREAM):
                cps.append(pltpu.async_copy(
                    row_v.at[b, j],
                    acc.at[idx_v.at[b, pl.ds(j * IDX_W, IDX_W)]],
                    sem_sc, add=True))
            for cp in cps:
                cp.wait()
            return carry

        lax.fori_loop(0, n_k, body, 0, unroll=False)
        plsc.subcore_barrier()

        pltpu.sync_copy(acc.at[pl.ds(s * STRIPE, STRIPE), :],
                        p_hbm.at[c, pl.ds(s * STRIPE, STRIPE), :])

    return k(recv, ft, zrows)


def _tc_body(x_r, w1_r, b1_r, wm2_r, bm2_r, wi2_r, bi2_r, wd2_r, bd2_r,
             p0_r, p1_r, dv_r, dw_r):
    x = x_r[...]
    h = jnp.maximum(
        jnp.dot(x, w1_r[...], preferred_element_type=jnp.float32) + b1_r[...],
        0.0)
    m = jnp.dot(h[:, :LATENT], wm2_r[...],
                preferred_element_type=jnp.float32) + bm2_r[...]
    i = jnp.dot(h[:, LATENT:2 * LATENT], wi2_r[...],
                preferred_element_type=jnp.float32) + bi2_r[...]
    d = jnp.dot(h[:, 2 * LATENT:], wd2_r[...],
                preferred_element_type=jnp.float32) + bd2_r[...]
    p = p0_r[...] + p1_r[...]
    f = p[:, 0:3]
    t = p[:, 4:7]
    dv_r[...] = m * f + d
    dw_r[...] = i * t


def _tc_decode(x, w1c, b1c, wm2, bm2, wi2, bi2, wd2, bd2, p0, p1):
    n = x.shape[0]
    blk = 4000
    grid = n // blk
    full = lambda shape: pl.BlockSpec(shape, lambda i: (0, 0))
    row = lambda w: pl.BlockSpec((blk, w), lambda i: (i, 0))
    return pl.pallas_call(
        _tc_body,
        grid=(grid,),
        in_specs=[
            row(LATENT),
            full((LATENT, 3 * LATENT)),
            full((1, 3 * LATENT)),
            full((LATENT, 1)),
            full((1, 1)),
            full((LATENT, 1)),
            full((1, 1)),
            full((LATENT, 3)),
            full((1, 3)),
            row(ROW_W), row(ROW_W),
        ],
        out_specs=[row(3), row(3)],
        out_shape=[
            jax.ShapeDtypeStruct((n, 3), jnp.float32),
            jax.ShapeDtypeStruct((n, 3), jnp.float32),
        ],
    )(x, w1c, b1c, wm2, bm2, wi2, bi2, wd2, bd2, p0, p1)


def kernel(edge_index, node_latent, fij, tij, Wm1, bm1, Wm2, bm2,
           Wi1, bi1, Wi2, bi2, Wd1, bd1, Wd2, bd2):
    n = node_latent.shape[0]
    recv = edge_index[1].astype(jnp.int32)
    zrows = jnp.zeros((STRIPE, ROW_W), jnp.float32)

    e = fij.shape[0]
    z1 = jnp.zeros((e, 1), jnp.float32)
    ft = jnp.concatenate([fij, z1, tij, z1], axis=1).reshape(e // IDX_W, IDX_W, ROW_W)
    p = _sc_scatter_partials(recv, ft, zrows)

    w1c = jnp.concatenate([Wm1, Wi1, Wd1], axis=1)
    b1c = jnp.concatenate([bm1, bi1, bd1]).reshape(1, 3 * LATENT)
    dv, dw = _tc_decode(
        node_latent, w1c, b1c,
        Wm2, bm2.reshape(1, 1), Wi2, bi2.reshape(1, 1),
        Wd2, bd2.reshape(1, 3),
        p[0, :n], p[1, :n])
    return (dv, dw)


# speedup vs baseline: 56.6358x; 3.1775x over previous
import functools

import jax
import jax.numpy as jnp
from jax import lax
from jax.experimental import pallas as pl
from jax.experimental.pallas import tpu as pltpu
from jax.experimental.pallas import tpu_sc as plsc

LATENT = 128
E_CHUNK = 1024
N_STREAM = 8
IDX_W = 128
ROW_W = 8
NC = 2
NS = 16
NW = NC * NS
STRIPE = 6250
N_PAD = STRIPE * NS


def _sc_scatter_partials(recv, planes, zrows):
    E = planes.shape[0] * IDX_W
    G = E // E_CHUNK
    SPC = E_CHUNK // IDX_W
    mesh = plsc.VectorSubcoreMesh(core_axis_name="c", subcore_axis_name="s")

    @functools.partial(
        pl.kernel,
        out_type=jax.ShapeDtypeStruct((NC, N_PAD, ROW_W), jnp.float32),
        mesh=mesh,
        scratch_types=[
            pltpu.VMEM_SHARED((N_PAD, ROW_W), jnp.float32),
            pltpu.VMEM((2, E_CHUNK), jnp.int32),
            pltpu.VMEM((2, SPC, ROW_W, IDX_W), jnp.float32),
            pltpu.VMEM((E_CHUNK, ROW_W), jnp.float32),
            pltpu.SemaphoreType.DMA,
            pltpu.SemaphoreType.DMA,
        ],
        compiler_params=pltpu.CompilerParams(
            use_tc_tiling_on_sc=False, needs_layout_passes=False),
    )
    def k(recv_hbm, pl_hbm, z_hbm, p_hbm, acc, idx_v, slab_v, comb_v,
          sem_in, sem_sc):
        c = lax.axis_index("c")
        s = lax.axis_index("s")
        tid = s * NC + c

        pltpu.sync_copy(z_hbm, acc.at[pl.ds(s * STRIPE, STRIPE), :])
        plsc.subcore_barrier()

        n_k = (G - tid + NW - 1) // NW

        lanes = lax.iota(jnp.int32, 16)
        rowc = [j8 * 16 + lanes for j8 in range(IDX_W // 16)]

        def stage(k_i, b):
            g = tid + k_i * NW
            pltpu.make_async_copy(
                recv_hbm.at[pl.ds(g * E_CHUNK, E_CHUNK)], idx_v.at[b],
                sem_in).start()
            pltpu.make_async_copy(
                pl_hbm.at[pl.ds(g * SPC, SPC), :, :], slab_v.at[b],
                sem_in).start()

        stage(0, 0)

        def body(k_i, carry):
            b = lax.rem(k_i, 2)
            pltpu.make_async_copy(
                recv_hbm.at[pl.ds(0, E_CHUNK)], idx_v.at[b], sem_in).wait()
            pltpu.make_async_copy(
                pl_hbm.at[pl.ds(0, SPC), :, :], slab_v.at[b], sem_in).wait()

            @pl.when(k_i + 1 < n_k)
            def _():
                stage(k_i + 1, 1 - b)

            def ileave(v, carry2):
                m_rel = v >> 3
                comp = v & 7
                cs = jnp.full((16,), comp, jnp.int32)
                base = m_rel * IDX_W
                for j8 in range(IDX_W // 16):
                    val = slab_v[b, m_rel, comp, pl.ds(j8 * 16, 16)]
                    plsc.store_scatter(comb_v, [base + rowc[j8], cs], val)
                return carry2

            lax.fori_loop(0, SPC * ROW_W, ileave, 0, unroll=False)

            cps = []
            for j in range(N_STREAM):
                cps.append(pltpu.async_copy(
                    comb_v.at[pl.ds(j * IDX_W, IDX_W), :],
                    acc.at[idx_v.at[b, pl.ds(j * IDX_W, IDX_W)]],
                    sem_sc, add=True))
            for cp in cps:
                cp.wait()
            return carry

        lax.fori_loop(0, n_k, body, 0, unroll=False)
        plsc.subcore_barrier()

        pltpu.sync_copy(acc.at[pl.ds(s * STRIPE, STRIPE), :],
                        p_hbm.at[c, pl.ds(s * STRIPE, STRIPE), :])

    return k(recv, planes, zrows)


def _tc_body(x_r, w1_r, b1_r, wm2_r, bm2_r, wi2_r, bi2_r, wd2_r, bd2_r,
             p_r, dv_r, dw_r):
    x = x_r[...]
    h = jnp.maximum(
        jnp.dot(x, w1_r[...], preferred_element_type=jnp.float32) + b1_r[...],
        0.0)
    m = jnp.dot(h[:, :LATENT], wm2_r[...],
                preferred_element_type=jnp.float32) + bm2_r[...]
    i = jnp.dot(h[:, LATENT:2 * LATENT], wi2_r[...],
                preferred_element_type=jnp.float32) + bi2_r[...]
    d = jnp.dot(h[:, 2 * LATENT:], wd2_r[...],
                preferred_element_type=jnp.float32) + bd2_r[...]
    p = p_r[0] + p_r[1]
    f = p[:, 0:3]
    t = p[:, 4:7]
    dv_r[...] = m * f + d
    dw_r[...] = i * t


def _tc_decode(x, w1c, b1c, wm2, bm2, wi2, bi2, wd2, bd2, p):
    n = x.shape[0]
    blk = 4000
    grid = n // blk
    full = lambda shape: pl.BlockSpec(shape, lambda i: (0, 0))
    row = lambda w: pl.BlockSpec((blk, w), lambda i: (i, 0))
    return pl.pallas_call(
        _tc_body,
        grid=(grid,),
        in_specs=[
            row(LATENT),
            full((LATENT, 3 * LATENT)),
            full((1, 3 * LATENT)),
            full((LATENT, 1)),
            full((1, 1)),
            full((LATENT, 1)),
            full((1, 1)),
            full((LATENT, 3)),
            full((1, 3)),
            pl.BlockSpec((NC, blk, ROW_W), lambda i: (0, i, 0)),
        ],
        out_specs=[row(3), row(3)],
        out_shape=[
            jax.ShapeDtypeStruct((n, 3), jnp.float32),
            jax.ShapeDtypeStruct((n, 3), jnp.float32),
        ],
    )(x, w1c, b1c, wm2, bm2, wi2, bi2, wd2, bd2, p)


def kernel(edge_index, node_latent, fij, tij, Wm1, bm1, Wm2, bm2,
           Wi1, bi1, Wi2, bi2, Wd1, bd1, Wd2, bd2):
    n = node_latent.shape[0]
    e = fij.shape[0]
    recv = edge_index[1].astype(jnp.int32)
    zrows = jnp.zeros((STRIPE, ROW_W), jnp.float32)

    z1 = jnp.zeros((e, 1), jnp.float32)
    packed = jnp.concatenate([fij, z1, tij, z1], axis=1)
    planes = jnp.transpose(packed.reshape(e // IDX_W, IDX_W, ROW_W),
                           (0, 2, 1))
    p = _sc_scatter_partials(recv, planes, zrows)

    w1c = jnp.concatenate([Wm1, Wi1, Wd1], axis=1)
    b1c = jnp.concatenate([bm1, bi1, bd1]).reshape(1, 3 * LATENT)
    dv, dw = _tc_decode(
        node_latent, w1c, b1c,
        Wm2, bm2.reshape(1, 1), Wi2, bi2.reshape(1, 1),
        Wd2, bd2.reshape(1, 3), p)
    return (dv, dw)


# speedup vs baseline: 59.8760x; 1.0572x over previous
import functools

import jax
import jax.numpy as jnp
from jax import lax
from jax.experimental import pallas as pl
from jax.experimental.pallas import tpu as pltpu
from jax.experimental.pallas import tpu_sc as plsc

LATENT = 128
E_CHUNK = 1024
N_STREAM = 8
IDX_W = 128
ROW_W = 8
NC = 2
NS = 16
NW = NC * NS
STRIPE = 6250
N_PAD = STRIPE * NS


def _sc_scatter_partials(recv, planes, zrows):
    E = planes.shape[0] * IDX_W
    G = E // E_CHUNK
    SPC = E_CHUNK // IDX_W
    mesh = plsc.VectorSubcoreMesh(core_axis_name="c", subcore_axis_name="s")

    @functools.partial(
        pl.kernel,
        out_type=jax.ShapeDtypeStruct((NC, N_PAD, ROW_W), jnp.float32),
        mesh=mesh,
        scratch_types=[
            pltpu.VMEM_SHARED((N_PAD, ROW_W), jnp.float32),
            pltpu.VMEM((4, E_CHUNK), jnp.int32),
            pltpu.VMEM((2, SPC, ROW_W, IDX_W), jnp.float32),
            pltpu.VMEM((2 * E_CHUNK, ROW_W), jnp.float32),
            pltpu.SemaphoreType.DMA,
            pltpu.SemaphoreType.DMA,
        ],
        compiler_params=pltpu.CompilerParams(
            use_tc_tiling_on_sc=False, needs_layout_passes=False),
    )
    def k(recv_hbm, pl_hbm, z_hbm, p_hbm, acc, idx_v, slab_v, comb_v,
          sem_in, sem_sc):
        c = lax.axis_index("c")
        s = lax.axis_index("s")
        tid = s * NC + c

        pltpu.sync_copy(z_hbm, acc.at[pl.ds(s * STRIPE, STRIPE), :])
        plsc.subcore_barrier()

        n_k = (G - tid + NW - 1) // NW

        lanes = lax.iota(jnp.int32, 16)
        rowc = [j8 * 16 + lanes for j8 in range(IDX_W // 16)]

        def stage(k_i):
            g = tid + k_i * NW
            b = lax.rem(k_i, 2)
            ib = lax.rem(k_i, 4)
            pltpu.make_async_copy(
                recv_hbm.at[pl.ds(g * E_CHUNK, E_CHUNK)], idx_v.at[ib],
                sem_in).start()
            pltpu.make_async_copy(
                pl_hbm.at[pl.ds(g * SPC, SPC), :, :], slab_v.at[b],
                sem_in).start()

        stage(0)

        def body(k_i, carry):
            b = lax.rem(k_i, 2)
            ib = lax.rem(k_i, 4)
            pltpu.make_async_copy(
                recv_hbm.at[pl.ds(0, E_CHUNK)], idx_v.at[ib], sem_in).wait()
            pltpu.make_async_copy(
                pl_hbm.at[pl.ds(0, SPC), :, :], slab_v.at[b], sem_in).wait()

            @pl.when(k_i >= 2)
            def _():
                for j in range(N_STREAM):
                    pltpu.make_async_copy(
                        comb_v.at[pl.ds(j * IDX_W, IDX_W), :],
                        acc.at[idx_v.at[ib, pl.ds(j * IDX_W, IDX_W)]],
                        sem_sc).wait()

            @pl.when(k_i + 1 < n_k)
            def _():
                stage(k_i + 1)

            cbase = b * E_CHUNK

            def ileave(v, carry2):
                m_rel = v >> 3
                comp = v & 7
                cs = jnp.full((16,), comp, jnp.int32)
                base = cbase + m_rel * IDX_W
                for j8 in range(IDX_W // 16):
                    val = slab_v[b, m_rel, comp, pl.ds(j8 * 16, 16)]
                    plsc.store_scatter(comb_v, [base + rowc[j8], cs], val)
                return carry2

            lax.fori_loop(0, SPC * ROW_W, ileave, 0, unroll=False)

            for j in range(N_STREAM):
                pltpu.async_copy(
                    comb_v.at[pl.ds(cbase + j * IDX_W, IDX_W), :],
                    acc.at[idx_v.at[ib, pl.ds(j * IDX_W, IDX_W)]],
                    sem_sc, add=True)
            return carry

        lax.fori_loop(0, n_k, body, 0, unroll=False)

        def drain(d, carry):
            for j in range(N_STREAM):
                pltpu.make_async_copy(
                    comb_v.at[pl.ds(j * IDX_W, IDX_W), :],
                    acc.at[idx_v.at[0, pl.ds(j * IDX_W, IDX_W)]],
                    sem_sc).wait()
            return carry

        lax.fori_loop(0, jnp.minimum(n_k, 2), drain, 0, unroll=False)
        plsc.subcore_barrier()

        pltpu.sync_copy(acc.at[pl.ds(s * STRIPE, STRIPE), :],
                        p_hbm.at[c, pl.ds(s * STRIPE, STRIPE), :])

    return k(recv, planes, zrows)


def _mlp_body(x_r, w1_r, b1_r, wm2_r, bm2_r, wi2_r, bi2_r, wd2_r, bd2_r,
              mid_r):
    x = x_r[...]
    h = jnp.maximum(
        jnp.dot(x, w1_r[...], preferred_element_type=jnp.float32) + b1_r[...],
        0.0)
    m = jnp.dot(h[:, :LATENT], wm2_r[...],
                preferred_element_type=jnp.float32) + bm2_r[...]
    i = jnp.dot(h[:, LATENT:2 * LATENT], wi2_r[...],
                preferred_element_type=jnp.float32) + bi2_r[...]
    d = jnp.dot(h[:, 2 * LATENT:], wd2_r[...],
                preferred_element_type=jnp.float32) + bd2_r[...]
    mid_r[...] = jnp.concatenate([m, i, d], axis=1)


def _tc_mlp(x, w1c, b1c, wm2, bm2, wi2, bi2, wd2, bd2):
    n = x.shape[0]
    blk = 4000
    grid = n // blk
    full = lambda shape: pl.BlockSpec(shape, lambda i: (0, 0))
    row = lambda w: pl.BlockSpec((blk, w), lambda i: (i, 0))
    return pl.pallas_call(
        _mlp_body,
        grid=(grid,),
        in_specs=[
            row(LATENT),
            full((LATENT, 3 * LATENT)),
            full((1, 3 * LATENT)),
            full((LATENT, 1)),
            full((1, 1)),
            full((LATENT, 1)),
            full((1, 1)),
            full((LATENT, 3)),
            full((1, 3)),
        ],
        out_specs=row(5),
        out_shape=jax.ShapeDtypeStruct((n, 5), jnp.float32),
    )(x, w1c, b1c, wm2, bm2, wi2, bi2, wd2, bd2)


def _combine_body(mid_r, p_r, dv_r, dw_r):
    mid = mid_r[...]
    p = p_r[0] + p_r[1]
    f = p[:, 0:3]
    t = p[:, 4:7]
    dv_r[...] = mid[:, 0:1] * f + mid[:, 2:5]
    dw_r[...] = mid[:, 1:2] * t


def _tc_combine(mid, p):
    n = mid.shape[0]
    blk = 4000
    grid = n // blk
    row = lambda w: pl.BlockSpec((blk, w), lambda i: (i, 0))
    return pl.pallas_call(
        _combine_body,
        grid=(grid,),
        in_specs=[
            row(5),
            pl.BlockSpec((NC, blk, ROW_W), lambda i: (0, i, 0)),
        ],
        out_specs=[row(3), row(3)],
        out_shape=[
            jax.ShapeDtypeStruct((n, 3), jnp.float32),
            jax.ShapeDtypeStruct((n, 3), jnp.float32),
        ],
    )(mid, p)


def kernel(edge_index, node_latent, fij, tij, Wm1, bm1, Wm2, bm2,
           Wi1, bi1, Wi2, bi2, Wd1, bd1, Wd2, bd2):
    n = node_latent.shape[0]
    e = fij.shape[0]
    recv = edge_index[1].astype(jnp.int32)
    zrows = jnp.zeros((STRIPE, ROW_W), jnp.float32)

    z1 = jnp.zeros((e, 1), jnp.float32)
    packed = jnp.concatenate([fij, z1, tij, z1], axis=1)
    planes = jnp.transpose(packed.reshape(e // IDX_W, IDX_W, ROW_W),
                           (0, 2, 1))
    p = _sc_scatter_partials(recv, planes, zrows)

    w1c = jnp.concatenate([Wm1, Wi1, Wd1], axis=1)
    b1c = jnp.concatenate([bm1, bi1, bd1]).reshape(1, 3 * LATENT)
    mid = _tc_mlp(
        node_latent, w1c, b1c,
        Wm2, bm2.reshape(1, 1), Wi2, bi2.reshape(1, 1),
        Wd2, bd2.reshape(1, 3))
    dv, dw = _tc_combine(mid, p)
    return (dv, dw)
